# baseline (device time: 226625 ns/iter reference)
import jax
import jax.numpy as jnp
from jax import lax
from jax.experimental import pallas as pl
from jax.experimental.pallas import tpu as pltpu

N_DEV = 4
KC = 2048
MH = 1024
NH = 2

_SLOT = (0, 1, 0, 1)


def kernel(x, w_mat):
    M, K = x.shape
    _, N = w_mat.shape
    NB = N // N_DEV
    NKC = K // KC

    my = lax.axis_index("i")
    offs = jnp.array([1, 2, 3, 0], dtype=jnp.int32)
    targets = (my + offs) % N_DEV

    def body(targ_ref, x_ref, w_ref, dummy_ref, out_ref,
             acc_ref, w_bf, send_bufs, send_sems, recv_sems, copy_sem):
        del dummy_ref
        t = pl.program_id(0)
        kc = pl.program_id(1)
        h = pl.program_id(2)
        my_pos = lax.axis_index("i")

        @pl.when((t == 0) & (kc == 0) & (h == 0))
        def _():
            barrier = pltpu.get_barrier_semaphore()
            for d in range(1, N_DEV):
                pl.semaphore_signal(
                    barrier, inc=1,
                    device_id=((my_pos + d) % N_DEV,),
                    device_id_type=pl.DeviceIdType.MESH,
                )
            pl.semaphore_wait(barrier, N_DEV - 1)

        @pl.when(h == 0)
        def _():
            w_bf[...] = w_ref[...].astype(jnp.bfloat16)

        prod = jnp.dot(x_ref[...].astype(jnp.bfloat16), w_bf[...],
                       preferred_element_type=jnp.float32)

        rows = pl.ds(h * MH, MH)

        @pl.when(kc == 0)
        def _():
            acc_ref[rows, :] = prod

        @pl.when(kc > 0)
        def _():
            acc_ref[rows, :] += prod

        def send_desc(r):
            return pltpu.make_async_remote_copy(
                src_ref=send_bufs.at[_SLOT[r]],
                dst_ref=out_ref.at[pl.ds(my_pos * M, M), :],
                send_sem=send_sems.at[_SLOT[r]],
                recv_sem=recv_sems.at[r],
                device_id=(targ_ref[r],),
                device_id_type=pl.DeviceIdType.MESH,
            )

        for tt in range(N_DEV):
            for hh in range(NH):
                @pl.when((kc == NKC - 1) & (t == tt) & (h == hh))
                def _(tt=tt, hh=hh):
                    slot = _SLOT[tt]
                    if hh == 0 and tt >= 2:
                        send_desc(tt - 2).wait_send()
                    y = jax.nn.gelu(
                        acc_ref[pl.ds(hh * MH, MH), :], approximate=True)
                    send_bufs[slot, pl.ds(hh * MH, MH), :] = (
                        y.astype(jnp.bfloat16))

                    if hh == NH - 1:
                        if tt < N_DEV - 1:
                            send_desc(tt).start()
                        else:
                            own_copy = pltpu.make_async_copy(
                                send_bufs.at[slot],
                                out_ref.at[pl.ds(my_pos * M, M), :],
                                copy_sem,
                            )
                            own_copy.start()

                            for r in range(N_DEV - 1):
                                src = (my_pos - (r + 1)) % N_DEV
                                recv_desc = pltpu.make_async_remote_copy(
                                    src_ref=send_bufs.at[_SLOT[r]],
                                    dst_ref=out_ref.at[pl.ds(src * M, M), :],
                                    send_sem=send_sems.at[_SLOT[r]],
                                    recv_sem=recv_sems.at[r],
                                    device_id=(my_pos,),
                                    device_id_type=pl.DeviceIdType.MESH,
                                )
                                recv_desc.wait_recv()

                            send_desc(2).wait_send()
                            own_copy.wait()

    grid_spec = pltpu.PrefetchScalarGridSpec(
        num_scalar_prefetch=1,
        grid=(N_DEV, NKC, NH),
        in_specs=[
            pl.BlockSpec((MH, KC), lambda t, kc, h, targ: (h, kc)),
            pl.BlockSpec((KC, NB), lambda t, kc, h, targ: (kc, targ[t])),
            pl.BlockSpec(memory_space=pl.ANY),
        ],
        out_specs=pl.BlockSpec(memory_space=pl.ANY),
        scratch_shapes=[
            pltpu.VMEM((M, NB), jnp.float32),
            pltpu.VMEM((KC, NB), jnp.bfloat16),
            pltpu.VMEM((2, M, NB), jnp.bfloat16),
            pltpu.SemaphoreType.DMA((2,)),
            pltpu.SemaphoreType.DMA((N_DEV - 1,)),
            pltpu.SemaphoreType.DMA,
        ],
    )

    dummy = pltpu.with_memory_space_constraint(
        jnp.zeros((N_DEV * M, NB), jnp.bfloat16), pltpu.MemorySpace.HBM)

    return pl.pallas_call(
        body,
        grid_spec=grid_spec,
        out_shape=jax.ShapeDtypeStruct((N_DEV * M, NB), jnp.bfloat16),
        input_output_aliases={3: 0},
        compiler_params=pltpu.CompilerParams(
            dimension_semantics=("arbitrary", "arbitrary", "arbitrary"),
            collective_id=0,
            vmem_limit_bytes=63 * 1024 * 1024,
        ),
    )(targets, x, w_mat, dummy)
